# baseline (speedup 1.0000x reference)
import functools

import jax
import jax.numpy as jnp
from jax import lax
from jax.experimental import pallas as pl
from jax.experimental.pallas import tpu as pltpu
from jax.experimental.pallas import tpu_sc as plsc

NUM_MONTHS = 12
N = 1_000_000
L = 16
NC = 2
NS = 16
NW = NC * NS
NV = N // L
BASE_V = NV // NW
EXTRA = NV - BASE_V * NW
CHUNK_V = 651
NCHUNKS = BASE_V // CHUNK_V
UNROLL = 7
ACC = 256
ACCB = 512


def _sc_body(out_hbm, tgt_hbm, ids_hbm, prt_hbm,
             obuf0, tbuf0, ibuf0, obuf1, tbuf1, ibuf1, sem0, sem1,
             accb):
    g = lax.axis_index("c") * NS + lax.axis_index("s")
    base_v = g * BASE_V + jnp.minimum(g, EXTRA)

    zeros = jnp.zeros((L,), jnp.float32)
    ones = jnp.ones((L,), jnp.float32)
    iota = lax.iota(jnp.int32, L)

    for j in range(ACCB // L):
        accb[pl.ds(j * L, L)] = zeros

    bufs = [(obuf0, tbuf0, ibuf0), (obuf1, tbuf1, ibuf1)]
    sems = [sem0, sem1]

    def start(c, b):
        eb = (base_v + c * CHUNK_V) * L
        sl = pl.ds(eb, CHUNK_V * L)
        ob, tb, ib = bufs[b]
        return [pltpu.async_copy(out_hbm.at[sl], ob, sems[b]),
                pltpu.async_copy(tgt_hbm.at[sl], tb, sems[b]),
                pltpu.async_copy(ids_hbm.at[sl], ib, sems[b])]

    pending = {0: start(0, 0)}
    for c in range(NCHUNKS):
        b = c & 1
        for h in pending.pop(b):
            h.wait()
        if c + 1 < NCHUNKS:
            pending[1 - b] = start(c + 1, 1 - b)
        ob, tb, ib = bufs[b]

        def load_block(blk):
            ids = [ib[pl.ds((blk * UNROLL + k) * L, L)] for k in range(UNROLL)]
            os_ = [ob[pl.ds((blk * UNROLL + k) * L, L)] for k in range(UNROLL)]
            ts_ = [tb[pl.ds((blk * UNROLL + k) * L, L)] for k in range(UNROLL)]
            idxs = tuple(ids[k] * L + iota for k in range(UNROLL))
            dfs = tuple(os_[k] - ts_[k] for k in range(UNROLL))
            return idxs + dfs

        def store_block(vals):
            for k in range(UNROLL):
                plsc.addupdate_scatter(accb, [vals[k]], vals[UNROLL + k])
                plsc.addupdate_scatter(accb, [vals[k] + ACC], ones)

        def vbody(blk, vals):
            nxt = load_block(blk + 1)
            store_block(vals)
            return nxt

        last = lax.fori_loop(0, CHUNK_V // UNROLL - 1, vbody, load_block(0))
        store_block(last)

    @pl.when(g < EXTRA)
    def _():
        eb = (base_v + BASE_V) * L
        sl = pl.ds(0, L)
        pltpu.sync_copy(ids_hbm.at[pl.ds(eb, L)], ibuf0.at[sl])
        pltpu.sync_copy(out_hbm.at[pl.ds(eb, L)], obuf0.at[sl])
        pltpu.sync_copy(tgt_hbm.at[pl.ds(eb, L)], tbuf0.at[sl])
        idx = ibuf0[sl] * L + iota
        plsc.addupdate_scatter(accb, [idx], obuf0[sl] - tbuf0[sl])
        plsc.addupdate_scatter(accb, [idx + ACC], ones)

    pltpu.sync_copy(accb, prt_hbm.at[g])


def _tc_finalize(prt_ref, out_ref):
    loss = jnp.float32(0.0)
    for m in range(NUM_MONTHS):
        sm = jnp.sum(prt_ref[:, pl.ds(m * L, L)])
        cm = jnp.sum(prt_ref[:, pl.ds(ACC + m * L, L)])
        d = sm / jnp.maximum(cm, 1.0)
        loss = loss + d * d
    out_ref[0, 0] = loss * jnp.float32(1.0 / NUM_MONTHS)


@jax.jit
def _monthly_loss(output, target, month_ids):
    mesh = plsc.VectorSubcoreMesh(
        core_axis_name="c", subcore_axis_name="s", num_cores=NC,
        num_subcores=NS)
    run = pl.kernel(
        _sc_body,
        out_type=jax.ShapeDtypeStruct((NW, ACCB), jnp.float32),
        mesh=mesh,
        scratch_types=[
            pltpu.VMEM((CHUNK_V * L,), jnp.float32),
            pltpu.VMEM((CHUNK_V * L,), jnp.float32),
            pltpu.VMEM((CHUNK_V * L,), jnp.int32),
            pltpu.VMEM((CHUNK_V * L,), jnp.float32),
            pltpu.VMEM((CHUNK_V * L,), jnp.float32),
            pltpu.VMEM((CHUNK_V * L,), jnp.int32),
            pltpu.SemaphoreType.DMA,
            pltpu.SemaphoreType.DMA,
            pltpu.VMEM((ACCB,), jnp.float32),
        ],
        compiler_params=pltpu.CompilerParams(needs_layout_passes=False),
    )
    prt = run(output, target, month_ids)
    res = pl.pallas_call(
        _tc_finalize,
        out_shape=jax.ShapeDtypeStruct((1, 1), jnp.float32),
        in_specs=[pl.BlockSpec(memory_space=pltpu.MemorySpace.VMEM)],
        out_specs=pl.BlockSpec(memory_space=pltpu.MemorySpace.SMEM),
    )(prt)
    return res[0, 0]


def kernel(output, target, month_ids):
    return _monthly_loss(output, target, month_ids)
